# baseline (speedup 1.0000x reference)
import functools

import jax
import jax.numpy as jnp
from jax import lax
from jax.experimental import pallas as pl
from jax.experimental.pallas import tpu as pltpu
from jax.experimental.pallas import tpu_sc as plsc

B, C, H, W = 16, 16, 384, 384
NUM_EMB = 100000
NC, NS = 2, 16
WCHUNK = 16
CHUNK = WCHUNK * H
NCHUNKS = W // WCHUNK
BPW = B // NC
TOTAL = BPW * NCHUNKS
NPAIRS = TOTAL // 2


def _idx_body(f_ref, o_ref):
    x = f_ref[0, 0, :, :]
    t = jnp.transpose(x)
    o_ref[0] = (t * float(NUM_EMB)).astype(jnp.int32)


def _tbl_body(t_ref, o_ref):
    o_ref[...] = jnp.transpose(t_ref[...])


def _sc_gather_body(tblT_hbm, idx_hbm, out_hbm, tbl_v, idx0, idx1, out0,
                    out1, isem0, isem1, osem0, osem1):
    cid = lax.axis_index("c")
    sid = lax.axis_index("s")
    pltpu.sync_copy(tblT_hbm.at[sid], tbl_v)

    def coords(j):
        b = cid * BPW + j // NCHUNKS
        return b, (j % NCHUNKS) * WCHUNK

    def idx_start(j, iv, isem):
        b, w0 = coords(j)
        pltpu.async_copy(idx_hbm.at[b, pl.ds(w0, WCHUNK)], iv, isem)

    def idx_wait(iv, isem):
        pltpu.make_async_copy(idx_hbm.at[0, pl.ds(0, WCHUNK)], iv, isem).wait()

    def out_start(j, ov, osem):
        b, w0 = coords(j)
        pltpu.async_copy(ov, out_hbm.at[b, sid, pl.ds(w0, WCHUNK)], osem)

    def out_wait(ov, osem):
        pltpu.make_async_copy(ov, out_hbm.at[0, 0, pl.ds(0, WCHUNK)], osem).wait()

    def gather(iv, ov):
        @plsc.parallel_loop(0, WCHUNK, 1)
        def _(w):
            @plsc.parallel_loop(0, H, 16, unroll=8)
            def _(h):
                ov[w, pl.ds(h, 16)] = plsc.load_gather(
                    tbl_v, [iv[w, pl.ds(h, 16)]])

    idx_start(0, idx0, isem0)
    idx_start(1, idx1, isem1)

    def pair(p, carry):
        j0 = 2 * p
        j1 = j0 + 1
        idx_wait(idx0, isem0)
        pl.when(p > 0)(lambda: out_wait(out0, osem0))
        gather(idx0, out0)
        out_start(j0, out0, osem0)
        pl.when(p < NPAIRS - 1)(lambda: idx_start(j0 + 2, idx0, isem0))

        idx_wait(idx1, isem1)
        pl.when(p > 0)(lambda: out_wait(out1, osem1))
        gather(idx1, out1)
        out_start(j1, out1, osem1)
        pl.when(p < NPAIRS - 1)(lambda: idx_start(j1 + 2, idx1, isem1))
        return carry

    lax.fori_loop(0, NPAIRS, pair, 0)
    out_wait(out0, osem0)
    out_wait(out1, osem1)


@functools.cache
def _build_sc_gather():
    mesh = plsc.VectorSubcoreMesh(
        core_axis_name="c", subcore_axis_name="s", num_cores=NC, num_subcores=NS
    )
    return pl.kernel(
        _sc_gather_body,
        out_type=jax.ShapeDtypeStruct((B, C, W, H), jnp.float32),
        mesh=mesh,
        scratch_types=[
            pltpu.VMEM((NUM_EMB,), jnp.float32),
            pltpu.VMEM((WCHUNK, H), jnp.int32),
            pltpu.VMEM((WCHUNK, H), jnp.int32),
            pltpu.VMEM((WCHUNK, H), jnp.float32),
            pltpu.VMEM((WCHUNK, H), jnp.float32),
            pltpu.SemaphoreType.DMA,
            pltpu.SemaphoreType.DMA,
            pltpu.SemaphoreType.DMA,
            pltpu.SemaphoreType.DMA,
        ],
        compiler_params=pltpu.CompilerParams(
            needs_layout_passes=False, use_tc_tiling_on_sc=True
        ),
    )


def kernel(feature, table):
    idxT = pl.pallas_call(
        _idx_body,
        grid=(B,),
        in_specs=[pl.BlockSpec((1, 1, H, W), lambda b: (b, 0, 0, 0))],
        out_specs=pl.BlockSpec((1, W, H), lambda b: (b, 0, 0)),
        out_shape=jax.ShapeDtypeStruct((B, W, H), jnp.int32),
    )(feature)
    tblT = pl.pallas_call(
        _tbl_body,
        out_shape=jax.ShapeDtypeStruct((C, NUM_EMB), jnp.float32),
    )(table)
    return _build_sc_gather()(tblT, idxT)


# speedup vs baseline: 1.2900x; 1.2900x over previous
import functools

import jax
import jax.numpy as jnp
from jax import lax
from jax.experimental import pallas as pl
from jax.experimental.pallas import tpu as pltpu
from jax.experimental.pallas import tpu_sc as plsc

B, C, H, W = 16, 16, 384, 384
NUM_EMB = 100000
NC, NS = 2, 16
WCHUNK = 8
CHUNK = WCHUNK * H
NCHUNKS = W // WCHUNK
BPW = B // NC
TOTAL = BPW * NCHUNKS
NPAIRS = TOTAL // 2


def _idx_body(f_ref, o_ref):
    x = f_ref[0, 0, :, :]
    t = jnp.transpose(x)
    o_ref[0] = (t * float(NUM_EMB)).astype(jnp.int32)


def _tbl_body(t_ref, o_ref):
    o_ref[...] = jnp.transpose(t_ref[...])


def _sc_gather_body(tblT_hbm, idx_hbm, out_hbm, tbl_v, idx0, idx1, out0,
                    out1, shared, isem0, isem1, osem0, osem1, ssem):
    cid = lax.axis_index("c")
    sid = lax.axis_index("s")
    pltpu.sync_copy(tblT_hbm.at[sid], tbl_v)

    def stage_start(bi):
        pltpu.async_copy(idx_hbm.at[cid * BPW + bi], shared.at[bi % 2], ssem)

    def stage_wait():
        pltpu.make_async_copy(idx_hbm.at[0], shared.at[0], ssem).wait()

    def idx_start(nb, w0, iv, isem):
        pltpu.async_copy(shared.at[nb, pl.ds(w0, WCHUNK)], iv, isem)

    def idx_wait(iv, isem):
        pltpu.make_async_copy(
            shared.at[0, pl.ds(0, WCHUNK)], iv, isem).wait()

    def out_start(b, w0, ov, osem):
        pltpu.async_copy(ov, out_hbm.at[b, sid, pl.ds(w0, WCHUNK)], osem)

    def out_wait(ov, osem):
        pltpu.make_async_copy(ov, out_hbm.at[0, 0, pl.ds(0, WCHUNK)], osem).wait()

    def gather(iv, ov):
        @plsc.parallel_loop(0, WCHUNK, 1)
        def _(w):
            @plsc.parallel_loop(0, H, 16, unroll=8)
            def _(h):
                ov[w, pl.ds(h, 16)] = plsc.load_gather(
                    tbl_v, [iv[w, pl.ds(h, 16)]])

    def _stage0():
        stage_start(0)
        stage_wait()

    pl.when(sid == 0)(_stage0)

    for bi in range(BPW):
        nb = bi % 2
        b = cid * BPW + bi
        plsc.subcore_barrier()
        if bi + 1 < BPW:
            pl.when(sid == 0)(lambda: stage_start(bi + 1))
        idx_start(nb, 0, idx0, isem0)
        idx_start(nb, WCHUNK, idx1, isem1)

        def pair(p, carry):
            w0 = 2 * p * WCHUNK
            idx_wait(idx0, isem0)
            pl.when(p > 0)(lambda: out_wait(out0, osem0))
            gather(idx0, out0)
            out_start(b, w0, out0, osem0)
            pl.when(p < NCHUNKS // 2 - 1)(
                lambda: idx_start(nb, w0 + 2 * WCHUNK, idx0, isem0))

            idx_wait(idx1, isem1)
            pl.when(p > 0)(lambda: out_wait(out1, osem1))
            gather(idx1, out1)
            out_start(b, w0 + WCHUNK, out1, osem1)
            pl.when(p < NCHUNKS // 2 - 1)(
                lambda: idx_start(nb, w0 + 3 * WCHUNK, idx1, isem1))
            return carry

        lax.fori_loop(0, NCHUNKS // 2, pair, 0)
        out_wait(out0, osem0)
        out_wait(out1, osem1)
        if bi + 1 < BPW:
            pl.when(sid == 0)(stage_wait)


@functools.cache
def _build_sc_gather():
    mesh = plsc.VectorSubcoreMesh(
        core_axis_name="c", subcore_axis_name="s", num_cores=NC, num_subcores=NS
    )
    return pl.kernel(
        _sc_gather_body,
        out_type=jax.ShapeDtypeStruct((B, C, W, H), jnp.float32),
        mesh=mesh,
        scratch_types=[
            pltpu.VMEM((NUM_EMB,), jnp.float32),
            pltpu.VMEM((WCHUNK, H), jnp.int32),
            pltpu.VMEM((WCHUNK, H), jnp.int32),
            pltpu.VMEM((WCHUNK, H), jnp.float32),
            pltpu.VMEM((WCHUNK, H), jnp.float32),
            pltpu.VMEM_SHARED((2, W, H), jnp.int32),
            pltpu.SemaphoreType.DMA,
            pltpu.SemaphoreType.DMA,
            pltpu.SemaphoreType.DMA,
            pltpu.SemaphoreType.DMA,
            pltpu.SemaphoreType.DMA,
        ],
        compiler_params=pltpu.CompilerParams(
            needs_layout_passes=False, use_tc_tiling_on_sc=True
        ),
    )


def kernel(feature, table):
    idxT = pl.pallas_call(
        _idx_body,
        grid=(B,),
        in_specs=[pl.BlockSpec((1, 1, H, W), lambda b: (b, 0, 0, 0))],
        out_specs=pl.BlockSpec((1, W, H), lambda b: (b, 0, 0)),
        out_shape=jax.ShapeDtypeStruct((B, W, H), jnp.int32),
    )(feature)
    tblT = pl.pallas_call(
        _tbl_body,
        out_shape=jax.ShapeDtypeStruct((C, NUM_EMB), jnp.float32),
    )(table)
    return _build_sc_gather()(tblT, idxT)


# speedup vs baseline: 1.6737x; 1.2975x over previous
import functools

import jax
import jax.numpy as jnp
from jax import lax
from jax.experimental import pallas as pl
from jax.experimental.pallas import tpu as pltpu
from jax.experimental.pallas import tpu_sc as plsc

B, C, H, W = 16, 16, 384, 384
NUM_EMB = 100000
NC, NS = 2, 16
WCHUNK = 8
CHUNK = WCHUNK * H
NCHUNKS = W // WCHUNK
BPW = B // NC
TOTAL = BPW * NCHUNKS
NPAIRS = TOTAL // 2


def _idx_body(f_ref, o_ref):
    x = f_ref[0, 0, :, :]
    t = jnp.transpose(x)
    o_ref[0] = (t * float(NUM_EMB)).astype(jnp.int32)


def _tbl_body(t_ref, o_ref):
    o_ref[...] = jnp.transpose(t_ref[...])


def _sc_gather_body(tblT_hbm, idx_hbm, out_hbm, tbl_v, idx0, idx1, out0,
                    out1, shared, isem0, isem1, osem0, osem1, ssem):
    cid = lax.axis_index("c")
    sid = lax.axis_index("s")
    pltpu.sync_copy(tblT_hbm.at[sid], tbl_v)

    def stage_start(bi):
        pltpu.async_copy(idx_hbm.at[cid * BPW + bi], shared.at[bi % 2], ssem)

    def stage_wait():
        pltpu.make_async_copy(idx_hbm.at[0], shared.at[0], ssem).wait()

    def idx_start(nb, w0, iv, isem):
        pltpu.async_copy(shared.at[nb, pl.ds(w0, WCHUNK)], iv, isem)

    def idx_wait(iv, isem):
        pltpu.make_async_copy(
            shared.at[0, pl.ds(0, WCHUNK)], iv, isem).wait()

    def out_start(b, w0, ov, osem):
        pltpu.async_copy(ov, out_hbm.at[b, sid, pl.ds(w0, WCHUNK)], osem)

    def out_wait(ov, osem):
        pltpu.make_async_copy(ov, out_hbm.at[0, 0, pl.ds(0, WCHUNK)], osem).wait()

    def gather(iv, ov):
        @plsc.parallel_loop(0, WCHUNK, 1)
        def _(w):
            @plsc.parallel_loop(0, H, 16, unroll=8)
            def _(h):
                ov[w, pl.ds(h, 16)] = plsc.load_gather(
                    tbl_v, [iv[w, pl.ds(h, 16)]])

    def _stage0():
        stage_start(0)
        stage_wait()

    pl.when(sid == 0)(_stage0)

    for bi in range(BPW):
        nb = bi % 2
        b = cid * BPW + bi
        plsc.subcore_barrier()
        if bi + 1 < BPW:
            pl.when(sid == 0)(lambda: stage_start(bi + 1))
        idx_start(nb, 0, idx0, isem0)
        idx_start(nb, WCHUNK, idx1, isem1)

        def pair(p, carry):
            w0 = 2 * p * WCHUNK
            idx_wait(idx0, isem0)
            pl.when(p > 0)(lambda: out_wait(out0, osem0))
            gather(idx0, out0)
            out_start(b, w0, out0, osem0)
            pl.when(p < NCHUNKS // 2 - 1)(
                lambda: idx_start(nb, w0 + 2 * WCHUNK, idx0, isem0))

            idx_wait(idx1, isem1)
            pl.when(p > 0)(lambda: out_wait(out1, osem1))
            gather(idx1, out1)
            out_start(b, w0 + WCHUNK, out1, osem1)
            pl.when(p < NCHUNKS // 2 - 1)(
                lambda: idx_start(nb, w0 + 3 * WCHUNK, idx1, isem1))
            return carry

        lax.fori_loop(0, NCHUNKS // 2, pair, 0)
        out_wait(out0, osem0)
        out_wait(out1, osem1)
        if bi + 1 < BPW:
            pl.when(sid == 0)(stage_wait)


@functools.cache
def _build_sc_gather():
    mesh = plsc.VectorSubcoreMesh(
        core_axis_name="c", subcore_axis_name="s", num_cores=NC, num_subcores=NS
    )
    return pl.kernel(
        _sc_gather_body,
        out_type=jax.ShapeDtypeStruct((B, C, W, H), jnp.float32),
        mesh=mesh,
        scratch_types=[
            pltpu.VMEM((NUM_EMB,), jnp.float32),
            pltpu.VMEM((WCHUNK, H), jnp.int32),
            pltpu.VMEM((WCHUNK, H), jnp.int32),
            pltpu.VMEM((WCHUNK, H), jnp.float32),
            pltpu.VMEM((WCHUNK, H), jnp.float32),
            pltpu.VMEM_SHARED((2, W, H), jnp.int32),
            pltpu.SemaphoreType.DMA,
            pltpu.SemaphoreType.DMA,
            pltpu.SemaphoreType.DMA,
            pltpu.SemaphoreType.DMA,
            pltpu.SemaphoreType.DMA,
        ],
        compiler_params=pltpu.CompilerParams(
            needs_layout_passes=False, use_tc_tiling_on_sc=True
        ),
    )


def kernel(feature, table):
    idxT = (jnp.transpose(feature[:, 0], (0, 2, 1)) * float(NUM_EMB)).astype(jnp.int32)
    tblT = jnp.transpose(table)
    return _build_sc_gather()(tblT, idxT)
